# baseline (speedup 1.0000x reference)
import functools

import jax
import jax.numpy as jnp
from jax import lax
from jax.experimental import pallas as pl
from jax.experimental.pallas import tpu as pltpu
from jax.experimental.pallas import tpu_sc as plsc

_H = 64
_W = 64
_BATCH = 16
_RATIOS = [(4, 4), (3, 5), (5, 3), (6, 6), (5, 7), (7, 5), (8, 8), (6, 10), (10, 6)]
_N_LIST = [3, 2, 1]
_NROW = [_H - rh + 1 for (rh, _) in _RATIOS]
_NCOL = [_W - rw + 1 for (_, rw) in _RATIOS]
_NWIN = [a * b for a, b in zip(_NROW, _NCOL)]
_GROUP_OFF = [0, sum(_NWIN[0:3]), sum(_NWIN[0:6])]
_SEC_OFF = [
    [0, _NWIN[0], _NWIN[0] + _NWIN[1]],
    [0, _NWIN[3], _NWIN[3] + _NWIN[4]],
    [0, _NWIN[6], _NWIN[6] + _NWIN[7]],
]
_NEG = -1e30
_GSIZE = 3 * 64 * 64


def _pool_body(x_ref, o_ref):
    X = x_ref[0, 0]
    r0 = lax.broadcasted_iota(jnp.int32, (64, 64), 0)
    c0 = lax.broadcasted_iota(jnp.int32, (64, 64), 1)
    d_rc = r0 - c0
    d_cr = c0 - r0
    col_sums = {}
    for rw in sorted({w for (_, w) in _RATIOS}):
        band = ((d_rc >= 0) & (d_rc < rw)).astype(jnp.float32)
        col_sums[rw] = jnp.dot(X, band, preferred_element_type=jnp.float32,
                               precision=lax.Precision.HIGHEST)
    for r, (rh, rw) in enumerate(_RATIOS):
        band = ((d_cr >= 0) & (d_cr < rh)).astype(jnp.float32)
        P = jnp.dot(band, col_sums[rw], preferred_element_type=jnp.float32,
                    precision=lax.Precision.HIGHEST)
        P = P * jnp.float32(1.0 / (rh * rw))
        valid = (r0 <= _H - rh) & (c0 <= _W - rw)
        o_ref[0, r] = jnp.where(valid, P, _NEG)


def _pool_scores(x):
    return pl.pallas_call(
        _pool_body,
        grid=(_BATCH,),
        in_specs=[pl.BlockSpec((1, 1, 64, 64), lambda b: (b, 0, 0, 0))],
        out_specs=pl.BlockSpec((1, 9, 64, 64), lambda b: (b, 0, 0, 0)),
        out_shape=jax.ShapeDtypeStruct((_BATCH, 9, 64, 64), jnp.float32),
    )(x)


def _sel3(r, v0, v1, v2):
    return jnp.where(r == 0, v0, jnp.where(r == 1, v1, v2))


def _perm(v, idx):
    dn = lax.GatherDimensionNumbers(
        offset_dims=(), collapsed_slice_dims=(0,), start_index_map=(0,))
    return lax.gather(v, idx.reshape(16, 1), dn, (1,),
                      mode=lax.GatherScatterMode.PROMISE_IN_BOUNDS)


def _bfly_max(v, lane):
    for sh in (1, 2, 4, 8):
        v = jnp.maximum(v, _perm(v, lane ^ sh))
    return v


def _bfly_min(v, lane):
    for sh in (1, 2, 4, 8):
        v = jnp.minimum(v, _perm(v, lane ^ sh))
    return v


def _nms_one_group(buf, g, n_keep):
    rhs = [_RATIOS[3 * g + r][0] for r in range(3)]
    rws = [_RATIOS[3 * g + r][1] for r in range(3)]
    ncs = [_NCOL[3 * g + r] for r in range(3)]
    secs = [_SEC_OFF[g][r] for r in range(3)]
    lane = lax.broadcasted_iota(jnp.int32, (16,), 0)
    n_chunks = _GSIZE // 16

    results = []
    zero = jnp.zeros((16,), jnp.int32)
    pi = pj = py1 = px1 = pa = zero
    for k in range(n_keep):
        def body(t, carry, k=k, pi=pi, pj=pj, py1=py1, px1=px1, pa=pa):
            mv, ivec = carry
            base = t * 16
            p = base + lane
            sl = buf[pl.ds(base, 16)]
            if k > 0:
                r = p >> 12
                i = (p >> 6) & 63
                j = p & 63
                rh = _sel3(r, rhs[0], rhs[1], rhs[2])
                rw = _sel3(r, rws[0], rws[1], rws[2])
                ih = jnp.minimum(py1, i + rh) - jnp.maximum(pi, i)
                iw = jnp.minimum(px1, j + rw) - jnp.maximum(pj, j)
                inter = jnp.maximum(ih, 0) * jnp.maximum(iw, 0)
                supp = (5 * inter) > (pa + rh * rw)
                sl = jnp.where(supp, _NEG, sl)
                if k < n_keep - 1:
                    buf[pl.ds(base, 16)] = sl
            upd = sl > mv
            mv = jnp.where(upd, sl, mv)
            ivec = jnp.where(upd, p, ivec)
            return mv, ivec

        mv0 = jnp.full((16,), _NEG, jnp.float32)
        iv0 = jnp.zeros((16,), jnp.int32)
        mv, ivec = lax.fori_loop(0, n_chunks, body, (mv0, iv0),
                                 unroll=8 if k == 0 else 4)
        m = _bfly_max(mv, lane)
        cand = jnp.where(mv == m, ivec, jnp.int32(2**31 - 1))
        pidx = _bfly_min(cand, lane)
        r0 = pidx >> 12
        i0 = (pidx >> 6) & 63
        j0 = pidx & 63
        prh = _sel3(r0, rhs[0], rhs[1], rhs[2])
        prw = _sel3(r0, rws[0], rws[1], rws[2])
        pi, pj, py1, px1, pa = i0, j0, i0 + prh, j0 + prw, prh * prw
        gidx = (_GROUP_OFF[g]
                + _sel3(r0, secs[0], secs[1], secs[2])
                + i0 * _sel3(r0, ncs[0], ncs[1], ncs[2])
                + j0)
        results.append((gidx, m))
    return results


def _nms_sc(flat):
    mesh = plsc.VectorSubcoreMesh(core_axis_name="c", subcore_axis_name="s")

    @functools.partial(
        pl.kernel,
        mesh=mesh,
        out_type=(
            jax.ShapeDtypeStruct((32, 16), jnp.int32),
            jax.ShapeDtypeStruct((32, 16), jnp.float32),
        ),
        scratch_types=[
            pltpu.VMEM((_GSIZE,), jnp.float32),
            pltpu.VMEM((16,), jnp.int32),
            pltpu.VMEM((16,), jnp.float32),
        ],
    )
    def k(flat_hbm, idx_out, score_out, buf, iv_vmem, sv_vmem):
        wid = lax.axis_index("s") * 2 + lax.axis_index("c")
        lane = lax.broadcasted_iota(jnp.int32, (16,), 0)

        def emit(picks, row):
            iv = jnp.zeros((16,), jnp.int32)
            sv = jnp.zeros((16,), jnp.float32)
            for slot, (gidx, m) in enumerate(picks):
                iv = jnp.where(lane == slot, gidx, iv)
                sv = jnp.where(lane == slot, m, sv)
            iv_vmem[...] = iv
            sv_vmem[...] = sv
            pltpu.sync_copy(iv_vmem, idx_out.at[row])
            pltpu.sync_copy(sv_vmem, score_out.at[row])

        @pl.when(wid < 16)
        def _():
            s_idx = wid
            pltpu.sync_copy(flat_hbm.at[s_idx, pl.ds(0, _GSIZE)], buf)
            emit(_nms_one_group(buf, 0, 3), wid)

        @pl.when(wid >= 16)
        def _():
            s_idx = wid - 16
            pltpu.sync_copy(flat_hbm.at[s_idx, pl.ds(_GSIZE, _GSIZE)], buf)
            picks = _nms_one_group(buf, 1, 2)
            pltpu.sync_copy(flat_hbm.at[s_idx, pl.ds(2 * _GSIZE, _GSIZE)], buf)
            picks += _nms_one_group(buf, 2, 1)
            emit(picks, wid)

    return k(flat)


def kernel(x, proposalN):
    padded = _pool_scores(x)
    window_scores = jnp.concatenate(
        [padded[:, r, : _NROW[r], : _NCOL[r]].reshape(_BATCH, -1) for r in range(9)],
        axis=1,
    )
    idx_rows, score_rows = _nms_sc(padded.reshape(_BATCH, 9 * 4096))
    indices = jnp.concatenate([idx_rows[:16, :3], idx_rows[16:32, :3]], axis=1)
    scores = jnp.concatenate([score_rows[:16, :3], score_rows[16:32, :3]], axis=1)
    return indices, scores, window_scores


# speedup vs baseline: 1.2657x; 1.2657x over previous
import functools

import jax
import jax.numpy as jnp
from jax import lax
from jax.experimental import pallas as pl
from jax.experimental.pallas import tpu as pltpu
from jax.experimental.pallas import tpu_sc as plsc

_H = 64
_W = 64
_BATCH = 16
_RATIOS = [(4, 4), (3, 5), (5, 3), (6, 6), (5, 7), (7, 5), (8, 8), (6, 10), (10, 6)]
_N_LIST = [3, 2, 1]
_NROW = [_H - rh + 1 for (rh, _) in _RATIOS]
_NCOL = [_W - rw + 1 for (_, rw) in _RATIOS]
_NWIN = [a * b for a, b in zip(_NROW, _NCOL)]
_GROUP_OFF = [0, sum(_NWIN[0:3]), sum(_NWIN[0:6])]
_SEC_OFF = [
    [0, _NWIN[0], _NWIN[0] + _NWIN[1]],
    [0, _NWIN[3], _NWIN[3] + _NWIN[4]],
    [0, _NWIN[6], _NWIN[6] + _NWIN[7]],
]
_NEG = -1e30
_GSIZE = 3 * 64 * 64


def _hshift(a, d):
    return jnp.concatenate([a[:, d:], jnp.zeros((1024, d), jnp.float32)], axis=1)


def _vshift(a, d):
    return jnp.concatenate([a[d:, :], jnp.zeros((d, 64), jnp.float32)], axis=0)


def _wsum(base_tree, w, shift):
    def get(p):
        if p not in base_tree:
            half = get(p // 2)
            base_tree[p] = half + shift(half, p // 2)
        return base_tree[p]
    powers = [1 << b for b in range(3, -1, -1) if w & (1 << b)]
    acc = get(powers[0])
    off = powers[0]
    for p in powers[1:]:
        acc = acc + shift(get(p), off)
        off += p
    return acc


def _pool_body(x_ref, o_ref):
    X = x_ref[:, 0, :, :].reshape(1024, 64)
    ii = lax.broadcasted_iota(jnp.int32, (1024, 64), 0) & 63
    jj = lax.broadcasted_iota(jnp.int32, (1024, 64), 1)
    htree = {1: X}
    hsums = {}
    vtrees = {}
    for r, (rh, rw) in enumerate(_RATIOS):
        if rw not in hsums:
            hsums[rw] = _wsum(htree, rw, _hshift)
            vtrees[rw] = {1: hsums[rw]}
        P = _wsum(vtrees[rw], rh, _vshift) * jnp.float32(1.0 / (rh * rw))
        valid = (ii <= _H - rh) & (jj <= _W - rw)
        o_ref[:, r] = jnp.where(valid, P, _NEG).reshape(16, 64, 64)


def _pool_scores(x):
    return pl.pallas_call(
        _pool_body,
        out_shape=jax.ShapeDtypeStruct((_BATCH, 9, 64, 64), jnp.float32),
    )(x)


def _sel3(r, v0, v1, v2):
    return jnp.where(r == 0, v0, jnp.where(r == 1, v1, v2))


def _perm(v, idx):
    dn = lax.GatherDimensionNumbers(
        offset_dims=(), collapsed_slice_dims=(0,), start_index_map=(0,))
    return lax.gather(v, idx.reshape(16, 1), dn, (1,),
                      mode=lax.GatherScatterMode.PROMISE_IN_BOUNDS)


def _bfly_max(v, lane):
    for sh in (1, 2, 4, 8):
        v = jnp.maximum(v, _perm(v, lane ^ sh))
    return v


def _bfly_min(v, lane):
    for sh in (1, 2, 4, 8):
        v = jnp.minimum(v, _perm(v, lane ^ sh))
    return v


def _nms_one_group(buf3, g, n_keep):
    rhs = [_RATIOS[3 * g + r][0] for r in range(3)]
    rws = [_RATIOS[3 * g + r][1] for r in range(3)]
    ncs = [_NCOL[3 * g + r] for r in range(3)]
    secs = [_SEC_OFF[g][r] for r in range(3)]
    lane = lax.broadcasted_iota(jnp.int32, (16,), 0)

    results = []
    zero = jnp.zeros((16,), jnp.int32)
    pi = pj = py1 = px1 = pa = zero
    for k in range(n_keep):
        carry = (jnp.full((16,), _NEG, jnp.float32), jnp.zeros((16,), jnp.int32))
        for sec in range(3):
            rh_s, rw_s, area_s = rhs[sec], rws[sec], rhs[sec] * rws[sec]

            def row_body(row, carry, k=k, sec=sec, rh_s=rh_s, rw_s=rw_s,
                         area_s=area_s, pi=pi, pj=pj, py1=py1, px1=px1, pa=pa):
                mv, ivec = carry
                rbase = sec * 4096 + row * 64
                if k > 0:
                    ih = jnp.minimum(py1, row + rh_s) - jnp.maximum(pi, row)
                    ih = jnp.maximum(ih, 0)
                for c in range(4):
                    sl = buf3[sec, row, pl.ds(c * 16, 16)]
                    p = rbase + c * 16 + lane
                    if k > 0:
                        jv = c * 16 + lane
                        iw = jnp.minimum(px1, jv + rw_s) - jnp.maximum(pj, jv)
                        inter = ih * jnp.maximum(iw, 0)
                        supp = (5 * inter) > (pa + area_s)
                        sl = jnp.where(supp, _NEG, sl)
                        if k < n_keep - 1:
                            buf3[sec, row, pl.ds(c * 16, 16)] = sl
                    upd = sl > mv
                    mv = jnp.where(upd, sl, mv)
                    ivec = jnp.where(upd, p, ivec)
                return mv, ivec

            carry = lax.fori_loop(0, 64, row_body, carry, unroll=2)
        mv, ivec = carry
        m = _bfly_max(mv, lane)
        cand = jnp.where(mv == m, ivec, jnp.int32(2**31 - 1))
        pidx = _bfly_min(cand, lane)
        r0 = pidx >> 12
        i0 = (pidx >> 6) & 63
        j0 = pidx & 63
        prh = _sel3(r0, rhs[0], rhs[1], rhs[2])
        prw = _sel3(r0, rws[0], rws[1], rws[2])
        pi, pj, py1, px1, pa = i0, j0, i0 + prh, j0 + prw, prh * prw
        gidx = (_GROUP_OFF[g]
                + _sel3(r0, secs[0], secs[1], secs[2])
                + i0 * _sel3(r0, ncs[0], ncs[1], ncs[2])
                + j0)
        results.append((gidx, m))
    return results


def _nms_sc(padded):
    mesh = plsc.VectorSubcoreMesh(core_axis_name="c", subcore_axis_name="s")

    @functools.partial(
        pl.kernel,
        mesh=mesh,
        out_type=(
            jax.ShapeDtypeStruct((32, 16), jnp.int32),
            jax.ShapeDtypeStruct((32, 16), jnp.float32),
        ),
        scratch_types=[
            pltpu.VMEM((3, 64, 64), jnp.float32),
            pltpu.VMEM((16,), jnp.int32),
            pltpu.VMEM((16,), jnp.float32),
        ],
    )
    def k(flat_hbm, idx_out, score_out, buf, iv_vmem, sv_vmem):
        wid = lax.axis_index("s") * 2 + lax.axis_index("c")
        lane = lax.broadcasted_iota(jnp.int32, (16,), 0)

        def emit(picks, row):
            iv = jnp.zeros((16,), jnp.int32)
            sv = jnp.zeros((16,), jnp.float32)
            for slot, (gidx, m) in enumerate(picks):
                iv = jnp.where(lane == slot, gidx, iv)
                sv = jnp.where(lane == slot, m, sv)
            iv_vmem[...] = iv
            sv_vmem[...] = sv
            pltpu.sync_copy(iv_vmem, idx_out.at[row])
            pltpu.sync_copy(sv_vmem, score_out.at[row])

        @pl.when(wid < 16)
        def _():
            s_idx = wid
            pltpu.sync_copy(flat_hbm.at[s_idx, pl.ds(0, 3)], buf)
            emit(_nms_one_group(buf, 0, 3), wid)

        @pl.when(wid >= 16)
        def _():
            s_idx = wid - 16
            pltpu.sync_copy(flat_hbm.at[s_idx, pl.ds(3, 3)], buf)
            picks = _nms_one_group(buf, 1, 2)
            pltpu.sync_copy(flat_hbm.at[s_idx, pl.ds(6, 3)], buf)
            picks += _nms_one_group(buf, 2, 1)
            emit(picks, wid)

    return k(padded)


def kernel(x, proposalN):
    padded = _pool_scores(x)
    window_scores = jnp.concatenate(
        [padded[:, r, : _NROW[r], : _NCOL[r]].reshape(_BATCH, -1) for r in range(9)],
        axis=1,
    )
    idx_rows, score_rows = _nms_sc(padded)
    indices = jnp.concatenate([idx_rows[:16, :3], idx_rows[16:32, :3]], axis=1)
    scores = jnp.concatenate([score_rows[:16, :3], score_rows[16:32, :3]], axis=1)
    return indices, scores, window_scores


# speedup vs baseline: 1.6620x; 1.3131x over previous
import functools

import jax
import jax.numpy as jnp
from jax import lax
from jax.experimental import pallas as pl
from jax.experimental.pallas import tpu as pltpu
from jax.experimental.pallas import tpu_sc as plsc

_H = 64
_W = 64
_BATCH = 16
_RATIOS = [(4, 4), (3, 5), (5, 3), (6, 6), (5, 7), (7, 5), (8, 8), (6, 10), (10, 6)]
_N_LIST = [3, 2, 1]
_NROW = [_H - rh + 1 for (rh, _) in _RATIOS]
_NCOL = [_W - rw + 1 for (_, rw) in _RATIOS]
_NWIN = [a * b for a, b in zip(_NROW, _NCOL)]
_GROUP_OFF = [0, sum(_NWIN[0:3]), sum(_NWIN[0:6])]
_SEC_OFF = [
    [0, _NWIN[0], _NWIN[0] + _NWIN[1]],
    [0, _NWIN[3], _NWIN[3] + _NWIN[4]],
    [0, _NWIN[6], _NWIN[6] + _NWIN[7]],
]
_NEG = -1e30
_GSIZE = 3 * 64 * 64


def _hshift(a, d):
    return jnp.concatenate([a[:, d:], jnp.zeros((1024, d), jnp.float32)], axis=1)


def _vshift(a, d):
    return jnp.concatenate([a[d:, :], jnp.zeros((d, 64), jnp.float32)], axis=0)


def _wsum(base_tree, w, shift):
    def get(p):
        if p not in base_tree:
            half = get(p // 2)
            base_tree[p] = half + shift(half, p // 2)
        return base_tree[p]
    powers = [1 << b for b in range(3, -1, -1) if w & (1 << b)]
    acc = get(powers[0])
    off = powers[0]
    for p in powers[1:]:
        acc = acc + shift(get(p), off)
        off += p
    return acc


def _pool_body(x_ref, o_ref):
    X = x_ref[:, 0, :, :].reshape(1024, 64)
    ii = lax.broadcasted_iota(jnp.int32, (1024, 64), 0) & 63
    jj = lax.broadcasted_iota(jnp.int32, (1024, 64), 1)
    htree = {1: X}
    hsums = {}
    vtrees = {}
    for r, (rh, rw) in enumerate(_RATIOS):
        if rw not in hsums:
            hsums[rw] = _wsum(htree, rw, _hshift)
            vtrees[rw] = {1: hsums[rw]}
        P = _wsum(vtrees[rw], rh, _vshift) * jnp.float32(1.0 / (rh * rw))
        valid = (ii <= _H - rh) & (jj <= _W - rw)
        o_ref[:, r] = jnp.where(valid, P, _NEG).reshape(16, 64, 64)


def _pool_scores(x):
    return pl.pallas_call(
        _pool_body,
        out_shape=jax.ShapeDtypeStruct((_BATCH, 9, 64, 64), jnp.float32),
    )(x)


_REAL_OFF = [sum(_NWIN[:r]) for r in range(9)]


def _pack_body(p_ref, o_ref):
    for r in range(9):
        nr, nc = _NROW[r], _NCOL[r]
        off = _REAL_OFF[r]
        for i in range(nr):
            o_ref[:, pl.ds(off + i * nc, nc)] = p_ref[:, r, i, pl.ds(0, nc)]


def _pack_scores(padded):
    return pl.pallas_call(
        _pack_body,
        out_shape=jax.ShapeDtypeStruct((_BATCH, sum(_NWIN)), jnp.float32),
    )(padded)


def _sel3(r, v0, v1, v2):
    return jnp.where(r == 0, v0, jnp.where(r == 1, v1, v2))


def _perm(v, idx):
    dn = lax.GatherDimensionNumbers(
        offset_dims=(), collapsed_slice_dims=(0,), start_index_map=(0,))
    return lax.gather(v, idx.reshape(16, 1), dn, (1,),
                      mode=lax.GatherScatterMode.PROMISE_IN_BOUNDS)


def _bfly_max(v, lane):
    for sh in (1, 2, 4, 8):
        v = jnp.maximum(v, _perm(v, lane ^ sh))
    return v


def _bfly_min(v, lane):
    for sh in (1, 2, 4, 8):
        v = jnp.minimum(v, _perm(v, lane ^ sh))
    return v


def _nms_one_group(buf3, g, n_keep):
    rhs = [_RATIOS[3 * g + r][0] for r in range(3)]
    rws = [_RATIOS[3 * g + r][1] for r in range(3)]
    ncs = [_NCOL[3 * g + r] for r in range(3)]
    secs = [_SEC_OFF[g][r] for r in range(3)]
    lane = lax.broadcasted_iota(jnp.int32, (16,), 0)

    results = []
    zero = jnp.zeros((16,), jnp.int32)
    pi = pj = py1 = px1 = pa = zero
    for k in range(n_keep):
        carry = (jnp.full((16,), _NEG, jnp.float32), jnp.zeros((16,), jnp.int32))
        for sec in range(3):
            rh_s, rw_s, area_s = rhs[sec], rws[sec], rhs[sec] * rws[sec]

            def row_body(row, carry, k=k, sec=sec, rh_s=rh_s, rw_s=rw_s,
                         area_s=area_s, pi=pi, pj=pj, py1=py1, px1=px1, pa=pa):
                mv, ivec = carry
                rbase = sec * 4096 + row * 64
                if k > 0:
                    ih = jnp.minimum(py1, row + rh_s) - jnp.maximum(pi, row)
                    ih = jnp.maximum(ih, 0)
                for c in range(4):
                    sl = buf3[sec, row, pl.ds(c * 16, 16)]
                    p = rbase + c * 16 + lane
                    if k > 0:
                        jv = c * 16 + lane
                        iw = jnp.minimum(px1, jv + rw_s) - jnp.maximum(pj, jv)
                        inter = ih * jnp.maximum(iw, 0)
                        supp = (5 * inter) > (pa + area_s)
                        sl = jnp.where(supp, _NEG, sl)
                        if k < n_keep - 1:
                            buf3[sec, row, pl.ds(c * 16, 16)] = sl
                    upd = sl > mv
                    mv = jnp.where(upd, sl, mv)
                    ivec = jnp.where(upd, p, ivec)
                return mv, ivec

            carry = lax.fori_loop(0, 64, row_body, carry, unroll=2)
        mv, ivec = carry
        m = _bfly_max(mv, lane)
        cand = jnp.where(mv == m, ivec, jnp.int32(2**31 - 1))
        pidx = _bfly_min(cand, lane)
        r0 = pidx >> 12
        i0 = (pidx >> 6) & 63
        j0 = pidx & 63
        prh = _sel3(r0, rhs[0], rhs[1], rhs[2])
        prw = _sel3(r0, rws[0], rws[1], rws[2])
        pi, pj, py1, px1, pa = i0, j0, i0 + prh, j0 + prw, prh * prw
        gidx = (_GROUP_OFF[g]
                + _sel3(r0, secs[0], secs[1], secs[2])
                + i0 * _sel3(r0, ncs[0], ncs[1], ncs[2])
                + j0)
        results.append((gidx, m))
    return results


def _nms_sc(padded):
    mesh = plsc.VectorSubcoreMesh(core_axis_name="c", subcore_axis_name="s")

    @functools.partial(
        pl.kernel,
        mesh=mesh,
        out_type=(
            jax.ShapeDtypeStruct((32, 16), jnp.int32),
            jax.ShapeDtypeStruct((32, 16), jnp.float32),
        ),
        scratch_types=[
            pltpu.VMEM((3, 64, 64), jnp.float32),
            pltpu.VMEM((16,), jnp.int32),
            pltpu.VMEM((16,), jnp.float32),
        ],
    )
    def k(flat_hbm, idx_out, score_out, buf, iv_vmem, sv_vmem):
        wid = lax.axis_index("s") * 2 + lax.axis_index("c")
        lane = lax.broadcasted_iota(jnp.int32, (16,), 0)

        def emit(picks, row):
            iv = jnp.zeros((16,), jnp.int32)
            sv = jnp.zeros((16,), jnp.float32)
            for slot, (gidx, m) in enumerate(picks):
                iv = jnp.where(lane == slot, gidx, iv)
                sv = jnp.where(lane == slot, m, sv)
            iv_vmem[...] = iv
            sv_vmem[...] = sv
            pltpu.sync_copy(iv_vmem, idx_out.at[row])
            pltpu.sync_copy(sv_vmem, score_out.at[row])

        @pl.when(wid < 16)
        def _():
            s_idx = wid
            pltpu.sync_copy(flat_hbm.at[s_idx, pl.ds(0, 3)], buf)
            emit(_nms_one_group(buf, 0, 3), wid)

        @pl.when(wid >= 16)
        def _():
            s_idx = wid - 16
            pltpu.sync_copy(flat_hbm.at[s_idx, pl.ds(3, 3)], buf)
            picks = _nms_one_group(buf, 1, 2)
            pltpu.sync_copy(flat_hbm.at[s_idx, pl.ds(6, 3)], buf)
            picks += _nms_one_group(buf, 2, 1)
            emit(picks, wid)

    return k(padded)


def kernel(x, proposalN):
    padded = _pool_scores(x)
    window_scores = _pack_scores(padded)
    idx_rows, score_rows = _nms_sc(padded)
    indices = jnp.concatenate([idx_rows[:16, :3], idx_rows[16:32, :3]], axis=1)
    scores = jnp.concatenate([score_rows[:16, :3], score_rows[16:32, :3]], axis=1)
    return indices, scores, window_scores
